# baseline (speedup 1.0000x reference)
import functools
import math

import jax
import jax.numpy as jnp
import numpy as np
from jax import lax
from jax.experimental import pallas as pl
from jax.experimental.pallas import tpu as pltpu
from jax.experimental.pallas import tpu_sc as plsc

DIM = 64
SEQ = 200
BATCH = 4096
TOTAL = BATCH * SEQ

NC = 2
NS = 16
NW = NC * NS
ROWS_PER_W = TOTAL // NW
CHUNK = SEQ
N_CHUNKS = ROWS_PER_W // CHUNK
SUB_OFFS = (0, 104)
SUB_LENS = (104, 96)
LANES = 16
VPER = DIM // LANES


def _pos_encoding():
    pos = np.arange(SEQ, dtype=np.float32)[:, None]
    fill = pos * np.exp(
        -np.arange(0, DIM, 2, dtype=np.float32) * math.log(10000.0) / DIM
    )
    enc = np.zeros((SEQ, DIM), dtype=np.float32)
    enc[:, 0::2] = np.sin(fill)
    enc[:, 1::2] = np.cos(fill)
    return enc


_mesh = plsc.VectorSubcoreMesh(core_axis_name="c", subcore_axis_name="s")


@functools.partial(
    pl.kernel,
    out_type=jax.ShapeDtypeStruct((TOTAL, DIM), jnp.float32),
    mesh=_mesh,
    scratch_types=[
        pltpu.VMEM((SEQ, DIM), jnp.float32),
        pltpu.VMEM((CHUNK,), jnp.int32),
        pltpu.VMEM((CHUNK, DIM), jnp.float32),
        pltpu.SemaphoreType.DMA,
    ],
    compiler_params=pltpu.CompilerParams(use_tc_tiling_on_sc=False),
)
def _emb_kernel(x_hbm, table_hbm, pos_hbm, out_hbm, pos_v, idx_v, rows_v, sem):
    wid = lax.axis_index("s") * NC + lax.axis_index("c")
    base = wid * ROWS_PER_W
    pltpu.sync_copy(pos_hbm, pos_v)

    def chunk_body(g, carry):
        off = base + g * CHUNK
        pltpu.sync_copy(x_hbm.at[pl.ds(off, CHUNK)], idx_v)
        copies = [
            pltpu.async_copy(
                table_hbm.at[idx_v.at[pl.ds(o, n)]],
                rows_v.at[pl.ds(o, n)],
                sem,
            )
            for o, n in zip(SUB_OFFS, SUB_LENS)
        ]
        for c in copies:
            c.wait()

        def add_row(r, rcarry):
            for j in range(VPER):
                sl = pl.ds(j * LANES, LANES)
                rows_v[r, sl] = rows_v[r, sl] + pos_v[r, sl]
            return rcarry

        lax.fori_loop(0, CHUNK, add_row, 0)
        pltpu.sync_copy(rows_v, out_hbm.at[pl.ds(off, CHUNK)])
        return carry

    lax.fori_loop(0, N_CHUNKS, chunk_body, 0)


def kernel(x, emb_table):
    pos = jnp.asarray(_pos_encoding())
    x_flat = x.reshape(TOTAL).astype(jnp.int32)
    out = _emb_kernel(x_flat, emb_table, pos)
    return out.reshape(BATCH, SEQ, DIM)


# speedup vs baseline: 1.1964x; 1.1964x over previous
import functools
import math

import jax
import jax.numpy as jnp
import numpy as np
from jax import lax
from jax.experimental import pallas as pl
from jax.experimental.pallas import tpu as pltpu
from jax.experimental.pallas import tpu_sc as plsc

DIM = 64
SEQ = 200
BATCH = 4096
TOTAL = BATCH * SEQ

NC = 2
NS = 16
NW = NC * NS
ROWS_PER_W = TOTAL // NW
CHUNK = SEQ
N_CHUNKS = ROWS_PER_W // CHUNK
NBUF = 4
N_STEPS = N_CHUNKS // NBUF
SUB_OFFS = (0, 104)
SUB_LENS = (104, 96)
LANES = 16
VPER = DIM // LANES


def _pos_encoding():
    pos = np.arange(SEQ, dtype=np.float32)[:, None]
    fill = pos * np.exp(
        -np.arange(0, DIM, 2, dtype=np.float32) * math.log(10000.0) / DIM
    )
    enc = np.zeros((SEQ, DIM), dtype=np.float32)
    enc[:, 0::2] = np.sin(fill)
    enc[:, 1::2] = np.cos(fill)
    return enc


_mesh = plsc.VectorSubcoreMesh(core_axis_name="c", subcore_axis_name="s")


@functools.partial(
    pl.kernel,
    out_type=jax.ShapeDtypeStruct((TOTAL, DIM), jnp.float32),
    mesh=_mesh,
    scratch_types=[
        pltpu.VMEM((SEQ, DIM), jnp.float32),
        pltpu.VMEM((ROWS_PER_W,), jnp.int32),
        [pltpu.VMEM((CHUNK, DIM), jnp.float32) for _ in range(NBUF)],
        pltpu.SemaphoreType.DMA((NBUF,)),
        pltpu.SemaphoreType.DMA((NBUF,)),
    ],
    compiler_params=pltpu.CompilerParams(use_tc_tiling_on_sc=False),
)
def _emb_kernel(x_hbm, table_hbm, pos_hbm, out_hbm, pos_v, idx_v, rows, gsem, ssem):
    wid = lax.axis_index("s") * NC + lax.axis_index("c")
    base = wid * ROWS_PER_W
    pltpu.sync_copy(pos_hbm, pos_v)
    pltpu.sync_copy(x_hbm.at[pl.ds(base, ROWS_PER_W)], idx_v)

    def fire_gather(c, b):
        for o, n in zip(SUB_OFFS, SUB_LENS):
            ioff = pl.multiple_of(c * CHUNK + o, 8)
            pltpu.async_copy(
                table_hbm.at[idx_v.at[pl.ds(ioff, n)]],
                rows[b].at[pl.ds(o, n)],
                gsem.at[b],
            )

    def wait_gather(b):
        pltpu.make_async_copy(
            out_hbm.at[pl.ds(0, CHUNK)], rows[b], gsem.at[b]
        ).wait()

    def fire_store(c, b):
        off = pl.multiple_of(base + c * CHUNK, 8)
        pltpu.async_copy(rows[b], out_hbm.at[pl.ds(off, CHUNK)], ssem.at[b])

    def wait_store(b):
        pltpu.make_async_copy(
            rows[b], out_hbm.at[pl.ds(0, CHUNK)], ssem.at[b]
        ).wait()

    fire_gather(0, 0)

    def step(p, carry):
        for b in range(NBUF):
            c = p * NBUF + b
            nb = (b + 1) % NBUF
            if b < NBUF - 1:
                pl.when(p > 0)(lambda nb=nb: wait_store(nb))
                fire_gather(c + 1, nb)
            else:
                wait_store(nb)
                pl.when(p < N_STEPS - 1)(lambda c=c, nb=nb: fire_gather(c + 1, nb))
            wait_gather(b)
            buf = rows[b]

            @plsc.parallel_loop(0, CHUNK, unroll=4)
            def _(r):
                for j in range(VPER):
                    sl = pl.ds(j * LANES, LANES)
                    buf[r, sl] = buf[r, sl] + pos_v[r, sl]

            fire_store(c, b)
        return carry

    lax.fori_loop(0, N_STEPS, step, 0)
    for b in range(1, NBUF):
        wait_store(b)


def kernel(x, emb_table):
    pos = jnp.asarray(_pos_encoding())
    x_flat = x.reshape(TOTAL).astype(jnp.int32)
    out = _emb_kernel(x_flat, emb_table, pos)
    return out.reshape(BATCH, SEQ, DIM)
